# baseline (speedup 1.0000x reference)
import jax
import jax.numpy as jnp
from jax.experimental import pallas as pl
from jax.experimental.pallas import tpu as pltpu

N_CLS = 1000
N_CTX = 4
DIM = 512
CTX_LEN = 77
SUFFIX_LEN = CTX_LEN - 1 - N_CTX

G = 8


def _concat_kernel(pref_ref, ctx_ref, suf_ref, out_ref):
    out_ref[0:1, :, :] = jnp.transpose(pref_ref[...], (1, 0, 2))
    ctx = ctx_ref[...]
    out_ref[1:1 + N_CTX, :, :] = jnp.broadcast_to(ctx[:, None, :],
                                                  (N_CTX, G, DIM))
    out_ref[1 + N_CTX:, :, :] = jnp.transpose(suf_ref[...], (1, 0, 2))


def kernel(prefixs, ctx, suffixs):
    grid = (N_CLS // G,)
    out_t = pl.pallas_call(
        _concat_kernel,
        grid=grid,
        in_specs=[
            pl.BlockSpec((G, 1, DIM), lambda i: (i, 0, 0)),
            pl.BlockSpec((N_CTX, DIM), lambda i: (0, 0)),
            pl.BlockSpec((G, SUFFIX_LEN, DIM), lambda i: (i, 0, 0)),
        ],
        out_specs=pl.BlockSpec((CTX_LEN, G, DIM), lambda i: (0, i, 0)),
        out_shape=jax.ShapeDtypeStruct((CTX_LEN, N_CLS, DIM), jnp.float32),
        compiler_params=pltpu.CompilerParams(
            dimension_semantics=("arbitrary",),
        ),
    )(prefixs, ctx, suffixs)
    return jnp.transpose(out_t, (1, 0, 2))


# speedup vs baseline: 1.5102x; 1.5102x over previous
import jax
import jax.numpy as jnp
from jax.experimental import pallas as pl
from jax.experimental.pallas import tpu as pltpu

N_CLS = 1000
N_CTX = 4
DIM = 512
CTX_LEN = 77
SUFFIX_LEN = CTX_LEN - 1 - N_CTX

G = 64


def _concat_kernel(pref_ref, ctx_ref, suf_ref, out_ref):
    out_ref[0:1, :, :] = jnp.transpose(pref_ref[...], (1, 0, 2))
    ctx = ctx_ref[...]
    out_ref[1:1 + N_CTX, :, :] = jnp.broadcast_to(ctx[:, None, :],
                                                  (N_CTX, G, DIM))
    out_ref[1 + N_CTX:, :, :] = jnp.transpose(suf_ref[...], (1, 0, 2))


def kernel(prefixs, ctx, suffixs):
    grid = (pl.cdiv(N_CLS, G),)
    out_t = pl.pallas_call(
        _concat_kernel,
        grid=grid,
        in_specs=[
            pl.BlockSpec((G, 1, DIM), lambda i: (i, 0, 0)),
            pl.BlockSpec((N_CTX, DIM), lambda i: (0, 0)),
            pl.BlockSpec((G, SUFFIX_LEN, DIM), lambda i: (i, 0, 0)),
        ],
        out_specs=pl.BlockSpec((CTX_LEN, G, DIM), lambda i: (0, i, 0)),
        out_shape=jax.ShapeDtypeStruct((CTX_LEN, N_CLS, DIM), jnp.float32),
        compiler_params=pltpu.CompilerParams(
            dimension_semantics=("arbitrary",),
        ),
    )(prefixs, ctx, suffixs)
    return jnp.transpose(out_t, (1, 0, 2))


# speedup vs baseline: 1.5194x; 1.0061x over previous
import jax
import jax.numpy as jnp
from jax.experimental import pallas as pl
from jax.experimental.pallas import tpu as pltpu

N_CLS = 1000
N_CTX = 4
DIM = 512
CTX_LEN = 77
SUFFIX_LEN = CTX_LEN - 1 - N_CTX

G = 80


def _concat_kernel(pref_ref, ctx_ref, suf_ref, out_ref):
    out_ref[0:1, :, :] = jnp.transpose(pref_ref[...], (1, 0, 2))
    ctx = ctx_ref[...]
    out_ref[1:1 + N_CTX, :, :] = jnp.broadcast_to(ctx[:, None, :],
                                                  (N_CTX, G, DIM))
    out_ref[1 + N_CTX:, :, :] = jnp.transpose(suf_ref[...], (1, 0, 2))


def kernel(prefixs, ctx, suffixs):
    grid = (pl.cdiv(N_CLS, G),)
    out_t = pl.pallas_call(
        _concat_kernel,
        grid=grid,
        in_specs=[
            pl.BlockSpec((G, 1, DIM), lambda i: (i, 0, 0)),
            pl.BlockSpec((N_CTX, DIM), lambda i: (0, 0)),
            pl.BlockSpec((G, SUFFIX_LEN, DIM), lambda i: (i, 0, 0)),
        ],
        out_specs=pl.BlockSpec((CTX_LEN, G, DIM), lambda i: (0, i, 0)),
        out_shape=jax.ShapeDtypeStruct((CTX_LEN, N_CLS, DIM), jnp.float32),
        compiler_params=pltpu.CompilerParams(
            dimension_semantics=("arbitrary",),
        ),
    )(prefixs, ctx, suffixs)
    return jnp.transpose(out_t, (1, 0, 2))


# speedup vs baseline: 1.5234x; 1.0026x over previous
import jax
import jax.numpy as jnp
from jax.experimental import pallas as pl
from jax.experimental.pallas import tpu as pltpu

N_CLS = 1000
N_CTX = 4
DIM = 512
CTX_LEN = 77
SUFFIX_LEN = CTX_LEN - 1 - N_CTX

G = 88


def _concat_kernel(pref_ref, ctx_ref, suf_ref, out_ref):
    out_ref[0:1, :, :] = jnp.transpose(pref_ref[...], (1, 0, 2))
    ctx = ctx_ref[...]
    out_ref[1:1 + N_CTX, :, :] = jnp.broadcast_to(ctx[:, None, :],
                                                  (N_CTX, G, DIM))
    out_ref[1 + N_CTX:, :, :] = jnp.transpose(suf_ref[...], (1, 0, 2))


def kernel(prefixs, ctx, suffixs):
    grid = (pl.cdiv(N_CLS, G),)
    out_t = pl.pallas_call(
        _concat_kernel,
        grid=grid,
        in_specs=[
            pl.BlockSpec((G, 1, DIM), lambda i: (i, 0, 0)),
            pl.BlockSpec((N_CTX, DIM), lambda i: (0, 0)),
            pl.BlockSpec((G, SUFFIX_LEN, DIM), lambda i: (i, 0, 0)),
        ],
        out_specs=pl.BlockSpec((CTX_LEN, G, DIM), lambda i: (0, i, 0)),
        out_shape=jax.ShapeDtypeStruct((CTX_LEN, N_CLS, DIM), jnp.float32),
        compiler_params=pltpu.CompilerParams(
            dimension_semantics=("arbitrary",),
        ),
    )(prefixs, ctx, suffixs)
    return jnp.transpose(out_t, (1, 0, 2))
